# baseline (speedup 1.0000x reference)
import functools

import jax
import jax.numpy as jnp
from jax import lax
from jax.experimental import pallas as pl
from jax.experimental.pallas import tpu as pltpu
from jax.experimental.pallas import tpu_sc as plsc

VOCAB = 1000000
EMB_DIM = 64

_info = plsc.get_sparse_core_info()
_NC, _NS = _info.num_cores, _info.num_subcores
_NW = _NC * _NS

_CHUNK = 128


def _make_lookup(B, D):
    assert B % (_NW * _CHUNK) == 0
    b_per_w = B // _NW
    n_chunks = b_per_w // _CHUNK
    mesh = plsc.VectorSubcoreMesh(core_axis_name="c", subcore_axis_name="s")

    @functools.partial(
        pl.kernel,
        mesh=mesh,
        out_type=jax.ShapeDtypeStruct((B, D), jnp.float32),
        scratch_types=[
            pltpu.VMEM((_CHUNK,), jnp.int32),
            pltpu.VMEM((_CHUNK, D), jnp.float32),
            pltpu.SemaphoreType.DMA,
        ],
        compiler_params=pltpu.CompilerParams(use_tc_tiling_on_sc=False),
    )
    def lookup(idx_hbm, table_hbm, out_hbm, idx_v, rows_v, sem):
        wid = lax.axis_index("s") * _NC + lax.axis_index("c")
        base = wid * b_per_w

        def chunk(g, carry):
            off = base + g * _CHUNK
            pltpu.sync_copy(idx_hbm.at[pl.ds(off, _CHUNK)], idx_v)
            pltpu.async_copy(table_hbm.at[idx_v], rows_v, sem).wait()
            pltpu.sync_copy(rows_v, out_hbm.at[pl.ds(off, _CHUNK)])
            return carry

        lax.fori_loop(0, n_chunks, chunk, 0)

    return lookup


def kernel(x, table):
    B0, B1 = x.shape
    flat = x.reshape((B0 * B1,)).astype(jnp.int32)
    out = _make_lookup(B0 * B1, table.shape[1])(flat, table)
    return out.reshape((B0, B1, table.shape[1]))


# speedup vs baseline: 1.2010x; 1.2010x over previous
import functools

import jax
import jax.numpy as jnp
from jax import lax
from jax.experimental import pallas as pl
from jax.experimental.pallas import tpu as pltpu
from jax.experimental.pallas import tpu_sc as plsc

_info = plsc.get_sparse_core_info()
_NC, _NS = _info.num_cores, _info.num_subcores
_NW = _NC * _NS

_CHUNK = 128
_K = 4


def _make_lookup(B, D):
    assert B % (_NW * _CHUNK * _K * 2) == 0
    b_per_w = B // _NW
    n_chunks = b_per_w // _CHUNK
    n_sets = n_chunks // _K
    mesh = plsc.VectorSubcoreMesh(core_axis_name="c", subcore_axis_name="s")

    @functools.partial(
        pl.kernel,
        mesh=mesh,
        out_type=jax.ShapeDtypeStruct((B, D), jnp.float32),
        scratch_types=[
            pltpu.VMEM((n_chunks, _CHUNK), jnp.int32),
            pltpu.VMEM((2, _K, _CHUNK, D), jnp.float32),
            pltpu.SemaphoreType.DMA((2,)),
            pltpu.SemaphoreType.DMA((2,)),
        ],
        compiler_params=pltpu.CompilerParams(use_tc_tiling_on_sc=False),
    )
    def lookup(idx_hbm, table_hbm, out_hbm, idx_v, rows_v, gsem, ssem):
        wid = lax.axis_index("s") * _NC + lax.axis_index("c")
        base = wid * b_per_w

        pltpu.sync_copy(idx_hbm.at[pl.ds(wid * n_chunks, n_chunks)], idx_v)

        def start_gathers(s, p):
            for j in range(_K):
                pltpu.make_async_copy(
                    table_hbm.at[idx_v.at[s * _K + j]],
                    rows_v.at[p, j],
                    gsem.at[p],
                ).start()

        def wait_gathers(s, p):
            for j in range(_K):
                pltpu.make_async_copy(
                    table_hbm.at[idx_v.at[s * _K + j]],
                    rows_v.at[p, j],
                    gsem.at[p],
                ).wait()

        def start_stores(s, p):
            for j in range(_K):
                pltpu.make_async_copy(
                    rows_v.at[p, j],
                    out_hbm.at[pl.ds(base + (s * _K + j) * _CHUNK, _CHUNK)],
                    ssem.at[p],
                ).start()

        def wait_stores(s, p):
            for j in range(_K):
                pltpu.make_async_copy(
                    rows_v.at[p, j],
                    out_hbm.at[pl.ds(base + (s * _K + j) * _CHUNK, _CHUNK)],
                    ssem.at[p],
                ).wait()

        start_gathers(0, 0)
        start_gathers(1, 1)
        wait_gathers(0, 0)
        start_stores(0, 0)

        def body(i, carry):
            for parity in (1, 0):
                s = 2 * i + (1 if parity == 1 else 2)
                p = parity
                q = 1 - p
                wait_stores(s - 1, q)
                start_gathers(s + 1, q)
                wait_gathers(s, p)
                start_stores(s, p)
            return carry

        lax.fori_loop(0, (n_sets - 2) // 2, body, 0)

        s_last = n_sets - 1
        wait_stores(s_last - 1, 0)
        wait_gathers(s_last, 1)
        start_stores(s_last, 1)
        wait_stores(s_last, 1)

    return lookup


def kernel(x, table):
    B0, B1 = x.shape
    B = B0 * B1
    idx2d = x.reshape((B // _CHUNK, _CHUNK)).astype(jnp.int32)
    out = _make_lookup(B, table.shape[1])(idx2d, table)
    return out.reshape((B0, B1, table.shape[1]))


# speedup vs baseline: 1.3243x; 1.1027x over previous
import functools

import jax
import jax.numpy as jnp
from jax import lax
from jax.experimental import pallas as pl
from jax.experimental.pallas import tpu as pltpu
from jax.experimental.pallas import tpu_sc as plsc

_info = plsc.get_sparse_core_info()
_NC, _NS = _info.num_cores, _info.num_subcores
_NW = _NC * _NS


def _make_lookup(B0, B1, DP):
    assert B0 % _NW == 0
    n_iter = B0 // _NW
    b_per_w = n_iter * B1
    D = 64
    mesh = plsc.VectorSubcoreMesh(core_axis_name="c", subcore_axis_name="s")

    @functools.partial(
        pl.kernel,
        mesh=mesh,
        out_type=jax.ShapeDtypeStruct((B0, B1, D), jnp.float32),
        scratch_types=[
            pltpu.VMEM((b_per_w,), jnp.int32),
            pltpu.VMEM((2, B1, DP), jnp.float32),
            pltpu.VMEM((2, B1, D), jnp.float32),
            pltpu.SemaphoreType.DMA((2,)),
            pltpu.SemaphoreType.DMA((2,)),
        ],
    )
    def lookup(x_hbm, table_hbm, out_hbm, idx_v, rows_v, rows64_v, gsem, ssem):
        wid = lax.axis_index("s") * _NC + lax.axis_index("c")
        base = wid * n_iter

        pltpu.sync_copy(x_hbm.at[pl.ds(base * B1, b_per_w)], idx_v)

        def gather_parts(it, p):
            off = it * B1
            yield idx_v.at[pl.ds(off, 128)], rows_v.at[p, pl.ds(0, 128)]
            yield (
                idx_v.at[pl.ds(off + 128, B1 - 128)],
                rows_v.at[p, pl.ds(128, B1 - 128)],
            )

        def start_gathers(it, p):
            for isl, rsl in gather_parts(it, p):
                pltpu.make_async_copy(table_hbm.at[isl], rsl, gsem.at[p]).start()

        def wait_gathers(it, p):
            for isl, rsl in gather_parts(it, p):
                pltpu.make_async_copy(table_hbm.at[isl], rsl, gsem.at[p]).wait()

        def compact(p):
            def cbody(j, carry):
                for r in range(8):
                    for k in range(D // 16):
                        rows64_v[p, j * 8 + r, pl.ds(k * 16, 16)] = rows_v[
                            p, j * 8 + r, pl.ds(k * 16, 16)
                        ]
                return carry

            lax.fori_loop(0, B1 // 8, cbody, 0)

        def start_store(it, p):
            pltpu.make_async_copy(
                rows64_v.at[p], out_hbm.at[base + it], ssem.at[p]
            ).start()

        def wait_store(it, p):
            pltpu.make_async_copy(
                rows64_v.at[p], out_hbm.at[base + it], ssem.at[p]
            ).wait()

        def step(it, p):
            q = 1 - p
            wait_store(it - 1, q)
            start_gathers(it + 1, q)
            wait_gathers(it, p)
            compact(p)
            start_store(it, p)

        start_gathers(0, 0)
        start_gathers(1, 1)
        wait_gathers(0, 0)
        compact(0)
        start_store(0, 0)

        def body(i, carry):
            for p in (1, 0):
                step(2 * i + (1 if p == 1 else 2), p)
            return carry

        lax.fori_loop(0, (n_iter - 2) // 2, body, 0)

        it = n_iter - 1
        wait_store(it - 1, 0)
        wait_gathers(it, 1)
        compact(1)
        start_store(it, 1)
        wait_store(it, 1)

    return lookup


def kernel(x, table):
    B0, B1 = x.shape
    V, D = table.shape
    DP = 128
    tpad = jnp.pad(table, ((0, 0), (0, DP - D)))
    xflat = x.reshape((B0 * B1,)).astype(jnp.int32)
    return _make_lookup(B0, B1, DP)(xflat, tpad)
